# baseline (speedup 1.0000x reference)
import functools
import math

import jax
import jax.numpy as jnp
from jax import lax
from jax.experimental import pallas as pl
from jax.experimental.pallas import tpu as pltpu
from jax.experimental.pallas import tpu_sc as plsc

NUM_EMB = 1000000
DIM = 64
BATCH = 4096
SEQ = 200

NW = 32
NBLK = (NUM_EMB + 127) // 128
K1 = (NBLK + NW - 1) // NW
TRROWS = NUM_EMB // 2
BBLK = BATCH // 128


def _pos_embedding():
    position = jnp.arange(0, SEQ, dtype=jnp.float32)[:, None]
    div_term = jnp.arange(0, DIM, 2, dtype=jnp.float32)
    div_term = jnp.exp(div_term * (-math.log(10000.0) / DIM))
    pe = jnp.zeros((SEQ, DIM), dtype=jnp.float32)
    pe = pe.at[:, 0::2].set(jnp.sin(position * div_term))
    pe = pe.at[:, 1::2].set(jnp.cos(position * div_term))
    return pe


def kernel(data, table):
    info = plsc.get_sparse_core_info()
    nc, ns = info.num_cores, info.num_subcores
    assert nc * ns == NW

    tT = table.T
    idx4 = (data.astype(jnp.int32)
            .reshape(32, 128, 25, 8).transpose(2, 0, 3, 1))
    pe = _pos_embedding()

    mesh1 = plsc.VectorSubcoreMesh(core_axis_name="c", subcore_axis_name="s")

    @functools.partial(
        pl.kernel,
        mesh=mesh1,
        compiler_params=pltpu.CompilerParams(use_tc_tiling_on_sc=True,
                                             needs_layout_passes=False),
        out_type=jax.ShapeDtypeStruct((TRROWS, 128), jnp.float32),
        scratch_types=[
            pltpu.VMEM((64, 128), jnp.float32),
            pltpu.VMEM((64, 128), jnp.float32),
            pltpu.VMEM((64, 128), jnp.float32),
            pltpu.VMEM((64, 128), jnp.float32),
            pltpu.SemaphoreType.DMA,
            pltpu.SemaphoreType.DMA,
            pltpu.SemaphoreType.DMA,
            pltpu.SemaphoreType.DMA,
        ],
    )
    def detile(tT_hbm, tr_hbm, sb0, sb1, ob0, ob1, g0, g1, o0, o1):
        wid = lax.axis_index("s") * nc + lax.axis_index("c")
        dvecs = [jnp.arange(16, dtype=jnp.int32) + 16 * j for j in range(4)]

        def blk(k):
            return jnp.minimum(wid + k * NW, NBLK - 1)

        def start_in(k, sb, sem):
            pltpu.make_async_copy(
                tT_hbm.at[:, pl.ds(blk(k) * 128, 128)], sb, sem).start()

        def wait_in(k, sb, sem):
            pltpu.make_async_copy(
                tT_hbm.at[:, pl.ds(blk(k) * 128, 128)], sb, sem).wait()

        def start_out(k, ob, sem):
            b = blk(k)
            pltpu.make_async_copy(
                ob.at[pl.ds(0, 32)], tr_hbm.at[pl.ds(b * 64, 32)], sem).start()

            @pl.when(b < NBLK - 1)
            def _():
                pltpu.make_async_copy(
                    ob.at[pl.ds(32, 32)],
                    tr_hbm.at[pl.ds(b * 64 + 32, 32)], sem).start()

        def wait_out(k, ob, sem):
            b = blk(k)
            pltpu.make_async_copy(
                ob.at[pl.ds(0, 32)], tr_hbm.at[pl.ds(b * 64, 32)], sem).wait()

            @pl.when(b < NBLK - 1)
            def _():
                pltpu.make_async_copy(
                    ob.at[pl.ds(32, 32)],
                    tr_hbm.at[pl.ds(b * 64 + 32, 32)], sem).wait()

        def transpose(sb, ob):
            for t in range(128):
                ts = jnp.full((16,), t, jnp.int32)
                base = (t % 2) * 64
                for j in range(4):
                    v = plsc.load_gather(sb, [dvecs[j], ts])
                    ob[t // 2, pl.ds(base + 16 * j, 16)] = v

        start_in(0, sb0, g0)

        def unit(k, sb, ob, gsem, osem, nsb, nob, ngsem, nosem):
            @pl.when(k + 1 < K1)
            def _():
                @pl.when(k >= 1)
                def _():
                    wait_out(k - 1, nob, nosem)
                start_in(k + 1, nsb, ngsem)

            wait_in(k, sb, gsem)
            transpose(sb, ob)
            start_out(k, ob, osem)

        def outer(g, carry):
            unit(2 * g, sb0, ob0, g0, o0, sb1, ob1, g1, o1)
            unit(2 * g + 1, sb1, ob1, g1, o1, sb0, ob0, g0, o0)
            return carry

        lax.fori_loop(0, K1 // 2, outer, 0)
        unit(K1 - 1, sb0, ob0, g0, o0, sb1, ob1, g1, o1)

        wait_out(K1 - 2, ob1, o1)
        wait_out(K1 - 1, ob0, o0)

    tr = detile(tT)
    table_lin = tr.reshape(NUM_EMB, 64)

    mesh2 = plsc.VectorSubcoreMesh(core_axis_name="c", subcore_axis_name="s")

    @functools.partial(
        pl.kernel,
        mesh=mesh2,
        compiler_params=pltpu.CompilerParams(use_tc_tiling_on_sc=False,
                                             needs_layout_passes=False),
        out_type=jax.ShapeDtypeStruct((SEQ, 8, BBLK, 8, 128), jnp.float32),
        scratch_types=[
            pltpu.VMEM((128,), jnp.int32),
            pltpu.VMEM((128,), jnp.int32),
            pltpu.VMEM((128, 64), jnp.float32),
            pltpu.VMEM((128, 64), jnp.float32),
            pltpu.VMEM((8, 8, 128), jnp.float32),
            pltpu.VMEM((8, 8, 128), jnp.float32),
            pltpu.VMEM((SEQ, DIM), jnp.float32),
            pltpu.SemaphoreType.DMA,
            pltpu.SemaphoreType.DMA,
            pltpu.SemaphoreType.DMA,
            pltpu.SemaphoreType.DMA,
        ],
    )
    def gather_add(idx_hbm, tab_hbm, pe_hbm, out_hbm,
                   ix0, ix1, gb0, gb1, ob0, ob1, pe_v, g0, g1, o0, o1):
        w = lax.axis_index("s") * nc + lax.axis_index("c")
        pltpu.sync_copy(pe_hbm, pe_v)
        tvecs = [jnp.arange(16, dtype=jnp.int32) + 16 * g for g in range(8)]

        def load_idx(s, ix):
            pltpu.sync_copy(idx_hbm.at[s // 8, w, lax.rem(s, 8)], ix)

        def start_gather(ix, gb, sem):
            pltpu.make_async_copy(tab_hbm.at[ix], gb, sem).start()

        def wait_gather(ix, gb, sem):
            pltpu.make_async_copy(tab_hbm.at[ix], gb, sem).wait()

        def start_store(s, ob, sem):
            for dt in range(8):
                pltpu.make_async_copy(
                    ob.at[dt], out_hbm.at[s, dt, w], sem).start()

        def wait_store(s, ob, sem):
            for dt in range(8):
                pltpu.make_async_copy(
                    ob.at[dt], out_hbm.at[s, dt, w], sem).wait()

        def transpose_add(s, gb, ob):
            for j in range(4):
                pej = pe_v[s, pl.ds(16 * j, 16)]
                for dd in range(16):
                    d = 16 * j + dd
                    pvec = lax.gather(
                        pej, jnp.full((16, 1), dd, jnp.int32),
                        lax.GatherDimensionNumbers(
                            offset_dims=(), collapsed_slice_dims=(0,),
                            start_index_map=(0,)),
                        slice_sizes=(1,),
                        mode=lax.GatherScatterMode.PROMISE_IN_BOUNDS)
                    dsplat = jnp.full((16,), d, jnp.int32)
                    for g in range(8):
                        v = plsc.load_gather(gb, [tvecs[g], dsplat]) + pvec
                        ob[d // 8, d % 8, pl.ds(16 * g, 16)] = v

        def unit(s, ix, gb, ob, gsem, osem, nix, ngb, nob, ngsem, nosem):
            @pl.when(s + 1 < SEQ)
            def _():
                @pl.when(s >= 1)
                def _():
                    wait_store(s - 1, nob, nosem)
                load_idx(s + 1, nix)
                start_gather(nix, ngb, ngsem)

            wait_gather(ix, gb, gsem)
            transpose_add(s, gb, ob)
            start_store(s, ob, osem)

        load_idx(0, ix0)
        start_gather(ix0, gb0, g0)

        def outer(g, carry):
            unit(2 * g, ix0, gb0, ob0, g0, o0, ix1, gb1, ob1, g1, o1)
            unit(2 * g + 1, ix1, gb1, ob1, g1, o1, ix0, gb0, ob0, g0, o0)
            return carry

        lax.fori_loop(0, SEQ // 2, outer, 0)

        wait_store(SEQ - 2, ob0, o0)
        wait_store(SEQ - 1, ob1, o1)

    out5 = gather_add(idx4, table_lin, pe)
    return out5.transpose(2, 4, 0, 1, 3).reshape(BATCH, SEQ, DIM)


# speedup vs baseline: 1.4623x; 1.4623x over previous
import functools
import math

import jax
import jax.numpy as jnp
from jax import lax
from jax.experimental import pallas as pl
from jax.experimental.pallas import tpu as pltpu
from jax.experimental.pallas import tpu_sc as plsc

NUM_EMB = 1000000
DIM = 64
BATCH = 4096
SEQ = 200

NW = 32
NBLK = (NUM_EMB + 127) // 128
K1 = (NBLK + NW - 1) // NW
TRROWS = NUM_EMB // 2
BBLK = BATCH // 128


def _pos_embedding():
    position = jnp.arange(0, SEQ, dtype=jnp.float32)[:, None]
    div_term = jnp.arange(0, DIM, 2, dtype=jnp.float32)
    div_term = jnp.exp(div_term * (-math.log(10000.0) / DIM))
    pe = jnp.zeros((SEQ, DIM), dtype=jnp.float32)
    pe = pe.at[:, 0::2].set(jnp.sin(position * div_term))
    pe = pe.at[:, 1::2].set(jnp.cos(position * div_term))
    return pe


def kernel(data, table):
    info = plsc.get_sparse_core_info()
    nc, ns = info.num_cores, info.num_subcores
    assert nc * ns == NW

    tT = table.T
    idx4 = (data.astype(jnp.int32)
            .reshape(32, 128, 25, 8).transpose(2, 0, 3, 1))
    pe = _pos_embedding()

    mesh1 = plsc.VectorSubcoreMesh(core_axis_name="c", subcore_axis_name="s")

    @functools.partial(
        pl.kernel,
        mesh=mesh1,
        compiler_params=pltpu.CompilerParams(use_tc_tiling_on_sc=True,
                                             needs_layout_passes=False),
        out_type=jax.ShapeDtypeStruct((TRROWS, 128), jnp.float32),
        scratch_types=[
            pltpu.VMEM((64, 128), jnp.float32),
            pltpu.VMEM((64, 128), jnp.float32),
            pltpu.VMEM((64, 128), jnp.float32),
            pltpu.VMEM((64, 128), jnp.float32),
            pltpu.SemaphoreType.DMA,
            pltpu.SemaphoreType.DMA,
            pltpu.SemaphoreType.DMA,
            pltpu.SemaphoreType.DMA,
        ],
    )
    def detile(tT_hbm, tr_hbm, sb0, sb1, ob0, ob1, g0, g1, o0, o1):
        wid = lax.axis_index("s") * nc + lax.axis_index("c")
        dvecs = [jnp.arange(16, dtype=jnp.int32) + 16 * j for j in range(4)]

        def blk(k):
            return jnp.minimum(wid + k * NW, NBLK - 1)

        def start_in(k, sb, sem):
            pltpu.make_async_copy(
                tT_hbm.at[:, pl.ds(blk(k) * 128, 128)], sb, sem).start()

        def wait_in(k, sb, sem):
            pltpu.make_async_copy(
                tT_hbm.at[:, pl.ds(blk(k) * 128, 128)], sb, sem).wait()

        def start_out(k, ob, sem):
            b = blk(k)
            pltpu.make_async_copy(
                ob.at[pl.ds(0, 32)], tr_hbm.at[pl.ds(b * 64, 32)], sem).start()

            @pl.when(b < NBLK - 1)
            def _():
                pltpu.make_async_copy(
                    ob.at[pl.ds(32, 32)],
                    tr_hbm.at[pl.ds(b * 64 + 32, 32)], sem).start()

        def wait_out(k, ob, sem):
            b = blk(k)
            pltpu.make_async_copy(
                ob.at[pl.ds(0, 32)], tr_hbm.at[pl.ds(b * 64, 32)], sem).wait()

            @pl.when(b < NBLK - 1)
            def _():
                pltpu.make_async_copy(
                    ob.at[pl.ds(32, 32)],
                    tr_hbm.at[pl.ds(b * 64 + 32, 32)], sem).wait()

        def transpose(sb, ob):
            for t0 in range(0, 128, 4):
                vs = []
                for t in range(t0, t0 + 4):
                    ts = jnp.full((16,), t, jnp.int32)
                    for j in range(4):
                        vs.append(plsc.load_gather(sb, [dvecs[j], ts]))
                i = 0
                for t in range(t0, t0 + 4):
                    for j in range(4):
                        ob[t // 2, pl.ds((t % 2) * 64 + 16 * j, 16)] = vs[i]
                        i += 1

        start_in(0, sb0, g0)

        def unit(k, sb, ob, gsem, osem, nsb, nob, ngsem, nosem):
            @pl.when(k + 1 < K1)
            def _():
                @pl.when(k >= 1)
                def _():
                    wait_out(k - 1, nob, nosem)
                start_in(k + 1, nsb, ngsem)

            wait_in(k, sb, gsem)
            transpose(sb, ob)
            start_out(k, ob, osem)

        def outer(g, carry):
            unit(2 * g, sb0, ob0, g0, o0, sb1, ob1, g1, o1)
            unit(2 * g + 1, sb1, ob1, g1, o1, sb0, ob0, g0, o0)
            return carry

        lax.fori_loop(0, K1 // 2, outer, 0)
        unit(K1 - 1, sb0, ob0, g0, o0, sb1, ob1, g1, o1)

        wait_out(K1 - 2, ob1, o1)
        wait_out(K1 - 1, ob0, o0)

    tr = detile(tT)
    table_lin = tr.reshape(NUM_EMB, 64)

    mesh2 = plsc.VectorSubcoreMesh(core_axis_name="c", subcore_axis_name="s")

    @functools.partial(
        pl.kernel,
        mesh=mesh2,
        compiler_params=pltpu.CompilerParams(use_tc_tiling_on_sc=False,
                                             needs_layout_passes=False),
        out_type=jax.ShapeDtypeStruct((SEQ, 8, BBLK, 8, 128), jnp.float32),
        scratch_types=[
            pltpu.VMEM((128,), jnp.int32),
            pltpu.VMEM((128,), jnp.int32),
            pltpu.VMEM((128, 64), jnp.float32),
            pltpu.VMEM((128, 64), jnp.float32),
            pltpu.VMEM((8, 8, 128), jnp.float32),
            pltpu.VMEM((8, 8, 128), jnp.float32),
            pltpu.VMEM((SEQ, DIM), jnp.float32),
            pltpu.SemaphoreType.DMA,
            pltpu.SemaphoreType.DMA,
            pltpu.SemaphoreType.DMA,
            pltpu.SemaphoreType.DMA,
        ],
    )
    def gather_add(idx_hbm, tab_hbm, pe_hbm, out_hbm,
                   ix0, ix1, gb0, gb1, ob0, ob1, pe_v, g0, g1, o0, o1):
        w = lax.axis_index("s") * nc + lax.axis_index("c")
        pltpu.sync_copy(pe_hbm, pe_v)
        tvecs = [jnp.arange(16, dtype=jnp.int32) + 16 * g for g in range(8)]

        def load_idx(s, ix):
            pltpu.sync_copy(idx_hbm.at[s // 8, w, lax.rem(s, 8)], ix)

        def start_gather(ix, gb, sem):
            pltpu.make_async_copy(tab_hbm.at[ix], gb, sem).start()

        def wait_gather(ix, gb, sem):
            pltpu.make_async_copy(tab_hbm.at[ix], gb, sem).wait()

        def start_store(s, ob, sem):
            for dt in range(8):
                pltpu.make_async_copy(
                    ob.at[dt], out_hbm.at[s, dt, w], sem).start()

        def wait_store(s, ob, sem):
            for dt in range(8):
                pltpu.make_async_copy(
                    ob.at[dt], out_hbm.at[s, dt, w], sem).wait()

        def transpose_add(s, gb, ob):
            for j in range(4):
                pej = pe_v[s, pl.ds(16 * j, 16)]

                for dd in range(16):
                    d = 16 * j + dd
                    pvec = lax.gather(
                        pej, jnp.full((16, 1), dd, jnp.int32),
                        lax.GatherDimensionNumbers(
                            offset_dims=(), collapsed_slice_dims=(0,),
                            start_index_map=(0,)),
                        slice_sizes=(1,),
                        mode=lax.GatherScatterMode.PROMISE_IN_BOUNDS)
                    dsplat = jnp.full((16,), d, jnp.int32)
                    vs = [plsc.load_gather(gb, [tvecs[g], dsplat]) + pvec
                          for g in range(8)]
                    for g in range(8):
                        ob[d // 8, d % 8, pl.ds(16 * g, 16)] = vs[g]

        def unit(s, ix, gb, ob, gsem, osem, nix, ngb, nob, ngsem, nosem):
            @pl.when(s + 1 < SEQ)
            def _():
                @pl.when(s >= 1)
                def _():
                    wait_store(s - 1, nob, nosem)
                load_idx(s + 1, nix)
                start_gather(nix, ngb, ngsem)

            wait_gather(ix, gb, gsem)
            transpose_add(s, gb, ob)
            start_store(s, ob, osem)

        load_idx(0, ix0)
        start_gather(ix0, gb0, g0)

        def outer(g, carry):
            unit(2 * g, ix0, gb0, ob0, g0, o0, ix1, gb1, ob1, g1, o1)
            unit(2 * g + 1, ix1, gb1, ob1, g1, o1, ix0, gb0, ob0, g0, o0)
            return carry

        lax.fori_loop(0, SEQ // 2, outer, 0)

        wait_store(SEQ - 2, ob0, o0)
        wait_store(SEQ - 1, ob1, o1)

    out5 = gather_add(idx4, table_lin, pe)
    return out5.transpose(2, 4, 0, 1, 3).reshape(BATCH, SEQ, DIM)
